# Initial kernel scaffold; baseline (speedup 1.0000x reference)
import jax
import jax.numpy as jnp
from jax.experimental import pallas as pl


def kernel(data, segment_ids):
    raise NotImplementedError("write your pallas kernel here")


# speedup vs baseline: 1.4354x; 1.4354x over previous
import functools

import jax
import jax.numpy as jnp
from jax import lax
from jax.experimental import pallas as pl
from jax.experimental.pallas import tpu as pltpu
from jax.experimental.pallas import tpu_sc as plsc

N_ROWS = 320000
D = 128
NSEG = 10000
NUM_CORES = 2
NUM_SUBCORES = 16
NW = NUM_CORES * NUM_SUBCORES
SEG_PER_W = 320
OUT_PAD = NW * SEG_PER_W
CHUNK = 256
SEARCH_STEPS = 19
NEG_INF = float("-inf")


def _sc_body(data_hbm, ids_hbm, out_hbm, bs_v, ids_v, rows_v, local_out):
    c = lax.axis_index("c")
    s = lax.axis_index("s")
    w = c * NUM_SUBCORES + s
    s0 = pl.multiple_of(w * SEG_PER_W, SEG_PER_W)

    neg = jnp.full((16,), NEG_INF, dtype=jnp.float32)

    def init_body(i, _):
        for t in range(D // 16):
            local_out[i, pl.ds(t * 16, 16)] = neg
        return 0

    lax.fori_loop(0, SEG_PER_W + 1, init_body, 0)

    def search_body(_, st):
        lo_a, hi_a, lo_b, hi_b = st
        mid_a = (lo_a + hi_a) // 2
        mid_b = (lo_b + hi_b) // 2
        base_a = pl.multiple_of(jnp.minimum(mid_a & ~15, N_ROWS - 16), 16)
        base_b = pl.multiple_of(jnp.minimum(mid_b & ~15, N_ROWS - 16), 16)
        pltpu.sync_copy(ids_hbm.at[pl.ds(base_a, 16)], bs_v.at[0, pl.ds(0, 16)])
        pltpu.sync_copy(ids_hbm.at[pl.ds(base_b, 16)], bs_v.at[1, pl.ds(0, 16)])
        bs_v[0, pl.ds(16, 16)] = bs_v[0, pl.ds(0, 16)]
        bs_v[1, pl.ds(16, 16)] = bs_v[1, pl.ds(0, 16)]
        va = bs_v[0, pl.ds(mid_a - base_a, 16)][0]
        vb = bs_v[1, pl.ds(mid_b - base_b, 16)][0]
        act_a = lo_a < hi_a
        act_b = lo_b < hi_b
        ge_a = va >= s0
        ge_b = vb >= s0 + SEG_PER_W
        lo_a = jnp.where(act_a & ~ge_a, mid_a + 1, lo_a)
        hi_a = jnp.where(act_a & ge_a, mid_a, hi_a)
        lo_b = jnp.where(act_b & ~ge_b, mid_b + 1, lo_b)
        hi_b = jnp.where(act_b & ge_b, mid_b, hi_b)
        return (lo_a, hi_a, lo_b, hi_b)

    z = jnp.int32(0)
    n = jnp.int32(N_ROWS)
    lo, _, hi, _ = lax.fori_loop(0, SEARCH_STEPS, search_body, (z, n, z, n))

    k0 = lo // CHUNK
    k1 = (hi + CHUNK - 1) // CHUNK

    def chunk_body(k, carry):
        base = pl.multiple_of(k * CHUNK, CHUNK)
        pltpu.sync_copy(ids_hbm.at[pl.ds(base, CHUNK)], ids_v)
        pltpu.sync_copy(data_hbm.at[pl.ds(base, CHUNK), :], rows_v)

        def group_body(g, rc):
            prev = rc[0]
            accs = list(rc[1:])
            idvec = ids_v[pl.ds(pl.multiple_of(g * 16, 16), 16)]
            for r in range(16):
                i = idvec[r]
                changed = i != prev
                j = i - s0
                valid = (j >= 0) & (j < SEG_PER_W)
                jj = jnp.where(valid, j, SEG_PER_W)
                for t in range(D // 16):
                    row_t = rows_v[g * 16 + r, pl.ds(t * 16, 16)]
                    a = jnp.maximum(jnp.where(changed, neg, accs[t]), row_t)
                    local_out[jj, pl.ds(t * 16, 16)] = a
                    accs[t] = a
                prev = i
            return (prev,) + tuple(accs)

        return lax.fori_loop(0, CHUNK // 16, group_body, carry)

    init = (jnp.int32(-1),) + tuple(neg for _ in range(D // 16))
    lax.fori_loop(k0, k1, chunk_body, init)

    pltpu.sync_copy(local_out.at[pl.ds(0, SEG_PER_W)],
                    out_hbm.at[pl.ds(s0, SEG_PER_W)])


@jax.jit
def _segment_max_sc(data, segment_ids):
    mesh = plsc.VectorSubcoreMesh(core_axis_name="c", subcore_axis_name="s")
    f = pl.kernel(
        _sc_body,
        mesh=mesh,
        out_type=jax.ShapeDtypeStruct((OUT_PAD, D), jnp.float32),
        scratch_types=[
            pltpu.VMEM((2, 32), jnp.int32),
            pltpu.VMEM((CHUNK,), jnp.int32),
            pltpu.VMEM((CHUNK, D), jnp.float32),
            pltpu.VMEM((SEG_PER_W + 1, D), jnp.float32),
        ],
    )
    return f(data, segment_ids)


def kernel(data, segment_ids):
    out = _segment_max_sc(data, segment_ids)
    return out[:NSEG]


# speedup vs baseline: 5.3351x; 3.7169x over previous
import jax
import jax.numpy as jnp
from jax import lax
from jax.experimental import pallas as pl
from jax.experimental.pallas import tpu as pltpu
from jax.experimental.pallas import tpu_sc as plsc

N_ROWS = 320000
D = 128
NSEG = 10000
NUM_CORES = 2
NUM_SUBCORES = 16
NW = NUM_CORES * NUM_SUBCORES
SEG_PER_W = 320
OUT_PAD = NW * SEG_PER_W
CHUNK = 256
NCHUNKS = N_ROWS // CHUNK
GROUPS = CHUNK // 16
NT = D // 16
SEARCH_STEPS = 19
NEG_INF = float("-inf")


def _sc_body(data_hbm, ids_hbm, out_hbm, bs_v, ids_v, rows_v, local_out,
             sem_i0, sem_i1, sem_r0, sem_r1):
    c = lax.axis_index("c")
    s = lax.axis_index("s")
    w = c * NUM_SUBCORES + s
    s0 = pl.multiple_of(w * SEG_PER_W, SEG_PER_W)

    neg = jnp.full((16,), NEG_INF, dtype=jnp.float32)

    def init_body(i, _):
        for t in range(NT):
            local_out[i, pl.ds(t * 16, 16)] = neg
        return 0

    lax.fori_loop(0, SEG_PER_W + 1, init_body, 0)

    def search_body(_, st):
        lo_a, hi_a, lo_b, hi_b = st
        mid_a = (lo_a + hi_a) // 2
        mid_b = (lo_b + hi_b) // 2
        base_a = pl.multiple_of(jnp.minimum(mid_a & ~15, N_ROWS - 16), 16)
        base_b = pl.multiple_of(jnp.minimum(mid_b & ~15, N_ROWS - 16), 16)
        pltpu.sync_copy(ids_hbm.at[pl.ds(base_a, 16)], bs_v.at[0, pl.ds(0, 16)])
        pltpu.sync_copy(ids_hbm.at[pl.ds(base_b, 16)], bs_v.at[1, pl.ds(0, 16)])
        bs_v[0, pl.ds(16, 16)] = bs_v[0, pl.ds(0, 16)]
        bs_v[1, pl.ds(16, 16)] = bs_v[1, pl.ds(0, 16)]
        va = bs_v[0, pl.ds(mid_a - base_a, 16)][0]
        vb = bs_v[1, pl.ds(mid_b - base_b, 16)][0]
        act_a = lo_a < hi_a
        act_b = lo_b < hi_b
        ge_a = va >= s0
        ge_b = vb >= s0 + SEG_PER_W
        lo_a = jnp.where(act_a & ~ge_a, mid_a + 1, lo_a)
        hi_a = jnp.where(act_a & ge_a, mid_a, hi_a)
        lo_b = jnp.where(act_b & ~ge_b, mid_b + 1, lo_b)
        hi_b = jnp.where(act_b & ge_b, mid_b, hi_b)
        return (lo_a, hi_a, lo_b, hi_b)

    z = jnp.int32(0)
    n = jnp.int32(N_ROWS)
    lo, _, hi, _ = lax.fori_loop(0, SEARCH_STEPS, search_body, (z, n, z, n))

    k0 = lo // CHUNK
    k1 = (hi + CHUNK - 1) // CHUNK
    odd = (k1 - k0) & 1
    up = odd & jnp.where(k1 < NCHUNKS, 1, 0)
    k1 = k1 + up
    k0 = k0 - (odd - up)

    sems_i = (sem_i0, sem_i1)
    sems_r = (sem_r0, sem_r1)

    def copies(k, b):
        base = pl.multiple_of(k * CHUNK, CHUNK)
        return (
            pltpu.make_async_copy(ids_hbm.at[pl.ds(base, CHUNK)],
                                  ids_v.at[b], sems_i[b]),
            pltpu.make_async_copy(data_hbm.at[pl.ds(base, CHUNK), :],
                                  rows_v.at[b], sems_r[b]),
        )

    @pl.when(k0 < k1)
    def _():
        for cp in copies(k0, 0):
            cp.start()

    def loads(b, ridx):
        return [rows_v[b, ridx, pl.ds(t * 16, 16)] for t in range(NT)]

    def process(k, b, carry):
        for cp in copies(k, b):
            cp.wait()

        def group_body(g, gc):
            prev = gc[0]
            accs = list(gc[1:1 + NT])
            nxt = list(gc[1 + NT:1 + 2 * NT])
            nxt2 = list(gc[1 + 2 * NT:1 + 3 * NT])
            idvec = gc[1 + 3 * NT]
            gbase = pl.multiple_of(g * 16, 16)
            idv_next = ids_v[b, pl.ds(jnp.minimum(gbase + 16, CHUNK - 16), 16)]
            for r in range(16):
                row = nxt
                nxt = nxt2
                nidx = gbase + r + 2
                if r >= 14:
                    nidx = jnp.minimum(nidx, CHUNK - 1)
                nxt2 = loads(b, nidx)
                i = idvec[r]
                changed = i != prev
                j = i - s0
                valid = (j >= 0) & (j < SEG_PER_W)
                jj = jnp.where(valid, j, SEG_PER_W)
                for t in range(NT):
                    a = jnp.maximum(jnp.where(changed, neg, accs[t]), row[t])
                    local_out[jj, pl.ds(t * 16, 16)] = a
                    accs[t] = a
                prev = i
            return (prev,) + tuple(accs) + tuple(nxt) + tuple(nxt2) \
                + (idv_next,)

        gc = lax.fori_loop(0, GROUPS, group_body,
                           carry + tuple(loads(b, 0)) + tuple(loads(b, 1))
                           + (ids_v[b, pl.ds(0, 16)],))
        return gc[:1 + NT]

    init = (jnp.int32(-1),) + tuple(neg for _ in range(NT))

    npairs = (k1 - k0) // 2

    def pair_body(kk, carry):
        cur = carry
        for b in (0, 1):
            k = k0 + kk * 2 + b

            @pl.when(k + 1 < k1)
            def _():
                for cp in copies(k + 1, 1 - b):
                    cp.start()

            cur = process(k, b, cur)
        return cur

    lax.fori_loop(0, npairs, pair_body, init)

    obase = pl.multiple_of(s0, SEG_PER_W)
    pltpu.sync_copy(local_out.at[pl.ds(0, SEG_PER_W)],
                    out_hbm.at[pl.ds(obase, SEG_PER_W)])


@jax.jit
def _segment_max_sc(data, segment_ids):
    mesh = plsc.VectorSubcoreMesh(core_axis_name="c", subcore_axis_name="s")
    f = pl.kernel(
        _sc_body,
        mesh=mesh,
        out_type=jax.ShapeDtypeStruct((OUT_PAD, D), jnp.float32),
        scratch_types=[
            pltpu.VMEM((2, 32), jnp.int32),
            pltpu.VMEM((2, CHUNK), jnp.int32),
            pltpu.VMEM((2, CHUNK, D), jnp.float32),
            pltpu.VMEM((SEG_PER_W + 1, D), jnp.float32),
            pltpu.SemaphoreType.DMA,
            pltpu.SemaphoreType.DMA,
            pltpu.SemaphoreType.DMA,
            pltpu.SemaphoreType.DMA,
        ],
    )
    return f(data, segment_ids)


def kernel(data, segment_ids):
    out = _segment_max_sc(data, segment_ids)
    return out[:NSEG]
